# Initial kernel scaffold; baseline (speedup 1.0000x reference)
import jax
import jax.numpy as jnp
from jax.experimental import pallas as pl


def kernel(feats_struct, feats_semantic, edge_types, edge_index, centrality, params):
    raise NotImplementedError("write your pallas kernel here")


# speedup vs baseline: 8.8444x; 8.8444x over previous
import functools

import jax
import jax.numpy as jnp
from jax import lax
from jax.experimental import pallas as pl
from jax.experimental.pallas import tpu as pltpu
from jax.experimental.pallas import tpu_sc as plsc

NEG_SLOPE = 0.2
ALPHA = 0.5
NH = 8
CH = 2 * NH
HID = 64
EPS = 1e-9


def _dense_body(xs_ref, xm_ref, w1s_ref, w1m_ref, b1_ref, w2_ref, b2_ref,
                cv_ref, el_ref, er_ref, ft_ref):
    h1s = jnp.maximum(
        jnp.dot(xs_ref[...], w1s_ref[...], preferred_element_type=jnp.float32)
        + b1_ref[0][None, :], 0.0)
    h1m = jnp.maximum(
        jnp.dot(xm_ref[...], w1m_ref[...], preferred_element_type=jnp.float32)
        + b1_ref[1][None, :], 0.0)
    h1 = jnp.concatenate([h1s, h1m], axis=1)
    h = jnp.dot(h1, w2_ref[...], preferred_element_type=jnp.float32) \
        + b2_ref[0][None, :]
    ft = h * cv_ref[0][None, :]
    el_ref[...] = ft * cv_ref[1][None, :]
    er_ref[...] = ft * cv_ref[2][None, :]
    ft_ref[...] = ft


def _dense_call(xs, xm, w1s, w1m, b1, w2blk, b2, cv0, block):
    n = xs.shape[0]
    grid = (n // block,)
    spec_x = pl.BlockSpec((block, xs.shape[1]), lambda i: (i, 0))
    spec_full = lambda a: pl.BlockSpec(a.shape, lambda i: (0,) * a.ndim)
    spec_o = pl.BlockSpec((block, CH), lambda i: (i, 0))
    return pl.pallas_call(
        _dense_body,
        grid=grid,
        in_specs=[spec_x, spec_x, spec_full(w1s), spec_full(w1m),
                  spec_full(b1), spec_full(w2blk), spec_full(b2),
                  spec_full(cv0)],
        out_specs=[spec_o, spec_o, spec_o],
        out_shape=[jax.ShapeDtypeStruct((n, CH), jnp.float32)] * 3,
    )(xs, xm, w1s, w1m, b1, w2blk, b2, cv0)


def _eet_body(rel_ref, ae_ref, out_ref):
    out_ref[0] = jnp.dot(rel_ref[...], ae_ref[0],
                         preferred_element_type=jnp.float32)


def _eet_call(rel_emb, ae_cat):
    L, P, _ = ae_cat.shape
    R = rel_emb.shape[0]
    return pl.pallas_call(
        _eet_body,
        grid=(L,),
        in_specs=[pl.BlockSpec((R, P), lambda l: (0, 0)),
                  pl.BlockSpec((1, P, CH), lambda l: (l, 0, 0))],
        out_specs=pl.BlockSpec((1, R, CH), lambda l: (l, 0, 0)),
        out_shape=jax.ShapeDtypeStruct((L, R, CH), jnp.float32),
    )(rel_emb, ae_cat)


def _post_body(s_ref, num_ref, ft_ref, cv_ref, el_ref, er_ref, fto_ref):
    rst = num_ref[0] / (s_ref[0] + EPS) + ft_ref[...]
    h = jnp.where(rst > 0, rst, jnp.exp(rst) - 1.0)
    hs = jnp.mean(h[:, :NH], axis=1, keepdims=True)
    hm = jnp.mean(h[:, NH:], axis=1, keepdims=True)
    h2 = jnp.concatenate([jnp.repeat(hs, NH, 1), jnp.repeat(hm, NH, 1)], axis=1)
    ft = h2 * cv_ref[0][None, :]
    el_ref[...] = ft * cv_ref[1][None, :]
    er_ref[...] = ft * cv_ref[2][None, :]
    fto_ref[...] = ft


def _post_call(sn3, ft, cv1, block):
    n = ft.shape[0]
    grid = (n // block,)
    spec_s = pl.BlockSpec((1, block, CH), lambda i: (0, i, 0))
    spec_n = pl.BlockSpec((1, block, CH), lambda i: (1, i, 0))
    spec_f = pl.BlockSpec((block, CH), lambda i: (i, 0))
    spec_cv = pl.BlockSpec(cv1.shape, lambda i: (0, 0))
    return pl.pallas_call(
        _post_body,
        grid=grid,
        in_specs=[spec_s, spec_n, spec_f, spec_cv],
        out_specs=[spec_f, spec_f, spec_f],
        out_shape=[jax.ShapeDtypeStruct((n, CH), jnp.float32)] * 3,
    )(sn3, sn3, ft, cv1)


def _final_body(s_ref, num_ref, ft_ref, cent_ref, gb_ref, out_ref):
    rst = num_ref[0] / (s_ref[0] + EPS) + ft_ref[...]
    h = jnp.where(rst > 0, rst, jnp.exp(rst) - 1.0)
    logits = ALPHA * h[:, :NH] + (1.0 - ALPHA) * h[:, NH:]
    scale = cent_ref[...] * gb_ref[0][None, :] + gb_ref[1][None, :]
    v = jnp.mean(scale * logits, axis=1, keepdims=True)
    out_ref[...] = jnp.where(v >= 0, v, 0.01 * v)


def _final_call(sn3, ft, cent, gb, block):
    n = ft.shape[0]
    grid = (n // block,)
    spec_s = pl.BlockSpec((1, block, CH), lambda i: (0, i, 0))
    spec_n = pl.BlockSpec((1, block, CH), lambda i: (1, i, 0))
    spec_f = pl.BlockSpec((block, CH), lambda i: (i, 0))
    spec_c = pl.BlockSpec((block, 1), lambda i: (i, 0))
    spec_gb = pl.BlockSpec(gb.shape, lambda i: (0, 0))
    return pl.pallas_call(
        _final_body,
        grid=grid,
        in_specs=[spec_s, spec_n, spec_f, spec_c, spec_gb],
        out_specs=pl.BlockSpec((block, 1), lambda i: (i, 0)),
        out_shape=jax.ShapeDtypeStruct((n, 1), jnp.float32),
    )(sn3, sn3, ft, cent, gb)


def _pad_rows(n_nodes):
    return -(-n_nodes // 10240) * 10240


def _make_edge_call(n_nodes, n_edges):
    S = 128
    C = 256
    CB = C // S
    NCHUNK = n_edges // C
    NT = 16
    NP = _pad_rows(n_nodes)
    NPT = NP // NT
    NZ = NPT // C

    mesh = plsc.VectorSubcoreMesh(core_axis_name="c", subcore_axis_name="s",
                                  num_cores=2, num_subcores=NT)

    @functools.partial(
        pl.kernel,
        out_type=jax.ShapeDtypeStruct((2 * NP, CH), jnp.float32),
        mesh=mesh,
        compiler_params=pltpu.CompilerParams(use_tc_tiling_on_sc=False),
        scratch_types=[
            pltpu.VMEM((CB, S), jnp.int32),
            pltpu.VMEM((CB, S), jnp.int32),
            pltpu.VMEM((CB, S), jnp.int32),
            pltpu.VMEM((C, CH), jnp.float32),
            pltpu.VMEM((C, CH), jnp.float32),
            pltpu.VMEM((C, CH), jnp.float32),
            pltpu.VMEM((C, CH), jnp.float32),
            pltpu.VMEM((C, CH), jnp.float32),
            pltpu.VMEM_SHARED((NP, CH), jnp.float32),
            pltpu.SemaphoreType.DMA,
        ],
    )
    def edge_kernel(src_hbm, dst_hbm, typ_hbm, el_hbm, er_hbm, ft_hbm, eet_hbm,
                    out_hbm, src_v, dst_v, typ_v, a_v, b_v, c_v, f_v, o_v,
                    tab, sem):
        cid = lax.axis_index("c")
        sid = lax.axis_index("s")

        def _zero(j, _):
            o_v[j] = jnp.zeros((CH,), jnp.float32)
            return 0
        lax.fori_loop(0, C, _zero, 0)
        for k in range(NZ):
            off = pl.multiple_of(sid * NPT + k * C, 8)
            pltpu.sync_copy(o_v, tab.at[pl.ds(off, C)])
        plsc.subcore_barrier()

        def _chunk(i, _):
            g = sid + i * NT
            row = pl.multiple_of(g * CB, 8)
            pltpu.sync_copy(src_hbm.at[pl.ds(row, CB)], src_v)
            pltpu.sync_copy(dst_hbm.at[pl.ds(row, CB)], dst_v)
            pltpu.sync_copy(typ_hbm.at[pl.ds(row, CB)], typ_v)
            cps = []
            for t in range(CB):
                cps.append(pltpu.async_copy(
                    el_hbm.at[src_v.at[t]], a_v.at[pl.ds(t * S, S)], sem))
                cps.append(pltpu.async_copy(
                    er_hbm.at[dst_v.at[t]], b_v.at[pl.ds(t * S, S)], sem))
                cps.append(pltpu.async_copy(
                    eet_hbm.at[typ_v.at[t]], c_v.at[pl.ds(t * S, S)], sem))

            @pl.when(cid == 1)
            def _gather_feat():
                fps = []
                for t in range(CB):
                    fps.append(pltpu.async_copy(
                        ft_hbm.at[src_v.at[t]], f_v.at[pl.ds(t * S, S)], sem))
                for d in fps:
                    d.wait()
            for d in cps:
                d.wait()

            @pl.when(cid == 0)
            def _compute_s():
                def cj(j, _):
                    v = a_v[j] + b_v[j] + c_v[j]
                    v = jnp.where(v >= 0, v, v * NEG_SLOPE)
                    o_v[j] = jnp.exp(v)
                    return 0
                lax.fori_loop(0, C, cj, 0)

            @pl.when(cid == 1)
            def _compute_num():
                def cj(j, _):
                    v = a_v[j] + b_v[j] + c_v[j]
                    v = jnp.where(v >= 0, v, v * NEG_SLOPE)
                    o_v[j] = jnp.exp(v) * f_v[j]
                    return 0
                lax.fori_loop(0, C, cj, 0)

            for t in range(CB):
                pltpu.sync_copy(o_v.at[pl.ds(t * S, S)],
                                tab.at[dst_v.at[t]], add=True)
            return 0

        base_iters = NCHUNK // NT
        extra = NCHUNK - base_iters * NT
        niter = base_iters + jnp.where(sid < extra, 1, 0)
        lax.fori_loop(0, niter, _chunk, 0)
        plsc.subcore_barrier()

        src_off = pl.multiple_of(sid * NPT, 8)
        dst_off = pl.multiple_of(cid * NP + sid * NPT, 8)
        pltpu.sync_copy(tab.at[pl.ds(src_off, NPT)],
                        out_hbm.at[pl.ds(dst_off, NPT)])

    return edge_kernel


def kernel(feats_struct, feats_semantic, edge_types, edge_index, centrality,
           params):
    n = feats_struct.shape[0]
    e = edge_types.shape[0]
    in_dim = feats_struct.shape[1]

    w1s = params['w1_s'].transpose(1, 0, 2).reshape(in_dim, NH * HID)
    w1m = params['w1_m'].transpose(1, 0, 2).reshape(in_dim, NH * HID)
    b1 = jnp.stack([params['b1_s'].reshape(-1), params['b1_m'].reshape(-1)])
    eye = jnp.eye(CH, dtype=jnp.float32)
    w2d = jnp.concatenate([params['w2_s'][:, :, 0],
                           params['w2_m'][:, :, 0]], axis=0)
    w2blk = (eye[:, None, :] * w2d[:, :, None]).reshape(CH * HID, CH)
    b2 = jnp.concatenate([params['b2_s'][:, 0],
                          params['b2_m'][:, 0]])[None, :]
    cv = [jnp.stack([jnp.concatenate([params['fc_s'][l], params['fc_m'][l]]),
                     jnp.concatenate([params['al_s'][l], params['al_m'][l]]),
                     jnp.concatenate([params['ar_s'][l], params['ar_m'][l]])])
          for l in range(2)]
    ae_cat = jnp.concatenate([params['ae_s'], params['ae_m']], axis=2)
    gb = jnp.concatenate([params['gamma'], params['beta']])

    src2d = edge_index[0].reshape(e // 128, 128)
    dst2d = edge_index[1].reshape(e // 128, 128)
    typ2d = edge_types.reshape(e // 128, 128)
    cent = centrality[:, None]

    block = 2000
    eet = _eet_call(params['rel_emb'], ae_cat)
    el0, er0, ft0 = _dense_call(feats_struct, feats_semantic, w1s, w1m, b1,
                                w2blk, b2, cv[0], block)
    edge_call = _make_edge_call(n, e)
    np_rows = _pad_rows(n)
    sn0 = edge_call(src2d, dst2d, typ2d, el0, er0, ft0, eet[0])
    el1, er1, ft1 = _post_call(sn0.reshape(2, np_rows, CH), ft0, cv[1], block)
    sn1 = edge_call(src2d, dst2d, typ2d, el1, er1, ft1, eet[1])
    return _final_call(sn1.reshape(2, np_rows, CH), ft1, cent, gb, block)
